# Initial kernel scaffold; baseline (speedup 1.0000x reference)
import functools

import jax
import jax.numpy as jnp
from jax import lax
from jax.experimental import pallas as pl
from jax.experimental.pallas import tpu as pltpu
from jax.experimental.pallas import tpu_sc as plsc

NC = 2
NS = 16
CH = 80


def _deg_body(npt, rows_per_tile, dst_hbm, out_hbm, idx_v, ones_v, zeros_v,
              acc_sh):
    c = lax.axis_index("c")
    s = lax.axis_index("s")

    @pl.when(c == 0)
    def _():
        def fill_ones(i, carry):
            ones_v[pl.ds(i * 16, 16)] = jnp.full((16,), 1.0, jnp.float32)
            return carry
        lax.fori_loop(0, CH // 16, fill_ones, 0)

        def fill_zeros(i, carry):
            zeros_v[pl.ds(i * 16, 16)] = jnp.zeros((16,), jnp.float32)
            return carry
        lax.fori_loop(0, npt // 16, fill_zeros, 0)

        pltpu.sync_copy(zeros_v, acc_sh.at[pl.ds(s * npt, npt)])
        plsc.subcore_barrier()

        pltpu.sync_copy(dst_hbm.at[pl.ds(s * rows_per_tile, rows_per_tile)],
                        idx_v)

        def chunk(j, carry):
            pltpu.sync_copy(ones_v, acc_sh.at[idx_v.at[j]], add=True)
            return carry
        lax.fori_loop(0, rows_per_tile, chunk, 0)

        plsc.subcore_barrier()
        pltpu.sync_copy(acc_sh.at[pl.ds(s * npt, npt)],
                        out_hbm.at[pl.ds(s * npt, npt)])


def _make_deg(n_pad, e_rows):
    rows_per_tile = e_rows // NS
    npt = n_pad // NS
    body = functools.partial(_deg_body, npt, rows_per_tile)
    return pl.kernel(
        body,
        jax.ShapeDtypeStruct((n_pad,), jnp.float32),
        mesh=plsc.VectorSubcoreMesh(core_axis_name="c", subcore_axis_name="s"),
        scratch_types=[
            pltpu.VMEM((rows_per_tile, CH), jnp.int32),
            pltpu.VMEM((CH,), jnp.float32),
            pltpu.VMEM((npt,), jnp.float32),
            pltpu.VMEM_SHARED((n_pad,), jnp.float32),
        ],
    )


def _prop_body(d, npt, rpt, y_hbm, src_hbm, dst_hbm, out_hbm, sidx_v, didx_v,
               rows_v, zeros_v, acc_sh, sem):
    c = lax.axis_index("c")
    s = lax.axis_index("s")
    wid = c * NS + s

    nz = d // 16

    def fill_zeros(i, carry):
        zeros_v[i // nz, pl.ds((i % nz) * 16, 16)] = jnp.zeros((16,),
                                                               jnp.float32)
        return carry
    lax.fori_loop(0, 64 * nz, fill_zeros, 0)

    def zero_acc(j, carry):
        pltpu.sync_copy(zeros_v, acc_sh.at[pl.ds(s * npt + j * 64, 64)])
        return carry
    lax.fori_loop(0, npt // 64, zero_acc, 0)
    plsc.subcore_barrier()

    pltpu.sync_copy(src_hbm.at[pl.ds(wid * rpt, rpt)], sidx_v)
    pltpu.sync_copy(dst_hbm.at[pl.ds(wid * rpt, rpt)], didx_v)

    def chunk(j, carry):
        pltpu.async_copy(y_hbm.at[sidx_v.at[j]], rows_v, sem).wait()
        pltpu.sync_copy(rows_v, acc_sh.at[didx_v.at[j]], add=True)
        return carry
    lax.fori_loop(0, rpt, chunk, 0)

    plsc.subcore_barrier()
    pltpu.sync_copy(acc_sh.at[pl.ds(s * npt, npt)],
                    out_hbm.at[c, pl.ds(s * npt, npt)])


def _make_prop(n, d, n_pad, e_rows):
    rpt = e_rows // (NC * NS)
    npt = n_pad // NS
    body = functools.partial(_prop_body, d, npt, rpt)
    return pl.kernel(
        body,
        jax.ShapeDtypeStruct((NC, n_pad, d), jnp.float32),
        mesh=plsc.VectorSubcoreMesh(core_axis_name="c", subcore_axis_name="s"),
        scratch_types=[
            pltpu.VMEM((rpt, CH), jnp.int32),
            pltpu.VMEM((rpt, CH), jnp.int32),
            pltpu.VMEM((CH, d), jnp.float32),
            pltpu.VMEM((64, d), jnp.float32),
            pltpu.VMEM_SHARED((n_pad, d), jnp.float32),
            pltpu.SemaphoreType.DMA,
        ],
    )


def _tc_a_body(deg_ref, x_ref, w_ref, y_ref, dinv_ref):
    dinv = lax.rsqrt(deg_ref[...] + 1.0)
    y_ref[...] = jnp.dot(x_ref[...], w_ref[...],
                         preferred_element_type=jnp.float32) * dinv
    dinv_ref[...] = dinv


def _tc_b_body(p_ref, y1_ref, dinv_ref, b1_ref, w2_ref, y2_ref):
    dinv = dinv_ref[...]
    h = jnp.maximum(dinv * (p_ref[0] + p_ref[1] + y1_ref[...]) + b1_ref[...],
                    0.0)
    y2_ref[...] = jnp.dot(h, w2_ref[...],
                          preferred_element_type=jnp.float32) * dinv


def _tc_c_body(q_ref, y2_ref, dinv_ref, b2_ref, out_ref):
    z = dinv_ref[...] * (q_ref[0] + q_ref[1] + y2_ref[...]) + b2_ref[...]
    m = jnp.max(z, axis=1, keepdims=True)
    lse = jnp.log(jnp.sum(jnp.exp(z - m), axis=1, keepdims=True)) + m
    out_ref[...] = z - lse


def kernel(x, edge_index, W1, b1, W2, b2):
    n, f = x.shape
    nh = W1.shape[1]
    nc = W2.shape[1]
    e = edge_index.shape[1]
    n_pad = ((n + NS * 64 - 1) // (NS * 64)) * (NS * 64)
    e_rows = e // CH
    R = 1000
    grid = (n // R,)

    ei = edge_index.astype(jnp.int32)
    src2 = ei[0].reshape(e_rows, CH)
    dst2 = ei[1].reshape(e_rows, CH)

    deg = _make_deg(n_pad, e_rows)(dst2)
    deg2 = deg.reshape(n_pad, 1)

    y1, dinv = pl.pallas_call(
        _tc_a_body,
        grid=grid,
        in_specs=[
            pl.BlockSpec((R, 1), lambda i: (i, 0)),
            pl.BlockSpec((R, f), lambda i: (i, 0)),
            pl.BlockSpec((f, nh), lambda i: (0, 0)),
        ],
        out_specs=[
            pl.BlockSpec((R, nh), lambda i: (i, 0)),
            pl.BlockSpec((R, 1), lambda i: (i, 0)),
        ],
        out_shape=[
            jax.ShapeDtypeStruct((n, nh), jnp.float32),
            jax.ShapeDtypeStruct((n, 1), jnp.float32),
        ],
    )(deg2, x, W1)

    p = _make_prop(n, nh, n_pad, e_rows)(y1, src2, dst2)

    y2 = pl.pallas_call(
        _tc_b_body,
        grid=grid,
        in_specs=[
            pl.BlockSpec((NC, R, nh), lambda i: (0, i, 0)),
            pl.BlockSpec((R, nh), lambda i: (i, 0)),
            pl.BlockSpec((R, 1), lambda i: (i, 0)),
            pl.BlockSpec((1, nh), lambda i: (0, 0)),
            pl.BlockSpec((nh, nc), lambda i: (0, 0)),
        ],
        out_specs=pl.BlockSpec((R, nc), lambda i: (i, 0)),
        out_shape=jax.ShapeDtypeStruct((n, nc), jnp.float32),
    )(p, y1, dinv, b1.reshape(1, nh), W2)

    q = _make_prop(n, nc, n_pad, e_rows)(y2, src2, dst2)

    out = pl.pallas_call(
        _tc_c_body,
        grid=grid,
        in_specs=[
            pl.BlockSpec((NC, R, nc), lambda i: (0, i, 0)),
            pl.BlockSpec((R, nc), lambda i: (i, 0)),
            pl.BlockSpec((R, 1), lambda i: (i, 0)),
            pl.BlockSpec((1, nc), lambda i: (0, 0)),
        ],
        out_specs=pl.BlockSpec((R, nc), lambda i: (i, 0)),
        out_shape=jax.ShapeDtypeStruct((n, nc), jnp.float32),
    )(q, y2, dinv, b2.reshape(1, nc))

    return out


# speedup vs baseline: 19.2952x; 19.2952x over previous
import functools

import jax
import jax.numpy as jnp
from jax import lax
from jax.experimental import pallas as pl
from jax.experimental.pallas import tpu as pltpu
from jax.experimental.pallas import tpu_sc as plsc

NC = 2
NS = 16
CH = 80


def _deg_body(npt, rows_per_tile, dst_hbm, out_hbm, idx_v, ones_v, zeros_v,
              acc_sh):
    c = lax.axis_index("c")
    s = lax.axis_index("s")

    @pl.when(c == 0)
    def _():
        def fill_ones(i, carry):
            ones_v[pl.ds(i * 16, 16)] = jnp.full((16,), 1.0, jnp.float32)
            return carry
        lax.fori_loop(0, CH // 16, fill_ones, 0)

        def fill_zeros(i, carry):
            zeros_v[pl.ds(i * 16, 16)] = jnp.zeros((16,), jnp.float32)
            return carry
        lax.fori_loop(0, npt // 16, fill_zeros, 0)

        pltpu.sync_copy(zeros_v, acc_sh.at[pl.ds(s * npt, npt)])
        plsc.subcore_barrier()

        pltpu.sync_copy(dst_hbm.at[s], idx_v)

        def chunk(j, carry):
            pltpu.sync_copy(ones_v, acc_sh.at[idx_v.at[j]], add=True)
            return carry
        lax.fori_loop(0, rows_per_tile, chunk, 0)

        plsc.subcore_barrier()
        pltpu.sync_copy(acc_sh.at[pl.ds(s * npt, npt)],
                        out_hbm.at[pl.ds(s * npt, npt)])


def _make_deg(n_pad, e_rows):
    rows_per_tile = e_rows // NS
    npt = n_pad // NS
    body = functools.partial(_deg_body, npt, rows_per_tile)
    return pl.kernel(
        body,
        jax.ShapeDtypeStruct((n_pad,), jnp.float32),
        mesh=plsc.VectorSubcoreMesh(core_axis_name="c", subcore_axis_name="s"),
        scratch_types=[
            pltpu.VMEM((rows_per_tile, CH), jnp.int32),
            pltpu.VMEM((CH,), jnp.float32),
            pltpu.VMEM((npt,), jnp.float32),
            pltpu.VMEM_SHARED((n_pad,), jnp.float32),
        ],
    )


def _prop_body(d, npt, rpt, y_hbm, src_hbm, dst_hbm, out_hbm, sidx_v, didx_v,
               rows_v, zeros_v, acc_sh, sem):
    c = lax.axis_index("c")
    s = lax.axis_index("s")
    wid = c * NS + s

    nz = d // 16

    def fill_zeros(i, carry):
        zeros_v[i // nz, pl.ds((i % nz) * 16, 16)] = jnp.zeros((16,),
                                                               jnp.float32)
        return carry
    lax.fori_loop(0, 16 * nz, fill_zeros, 0)

    def zero_acc(j, carry):
        pltpu.sync_copy(zeros_v, acc_sh.at[pl.ds(s * npt + j * 16, 16)])
        return carry
    lax.fori_loop(0, npt // 16, zero_acc, 0)
    plsc.subcore_barrier()

    pltpu.sync_copy(src_hbm.at[wid], sidx_v)
    pltpu.sync_copy(dst_hbm.at[wid], didx_v)

    def chunk(j, carry):
        pltpu.async_copy(y_hbm.at[sidx_v.at[j]], rows_v, sem).wait()
        pltpu.sync_copy(rows_v, acc_sh.at[didx_v.at[j]], add=True)
        return carry
    lax.fori_loop(0, rpt, chunk, 0)

    plsc.subcore_barrier()
    pltpu.sync_copy(acc_sh.at[pl.ds(s * npt, npt)],
                    out_hbm.at[c, pl.ds(s * npt, npt)])


def _make_prop(n, d, n_pad, e_rows):
    rpt = e_rows // (NC * NS)
    npt = n_pad // NS
    body = functools.partial(_prop_body, d, npt, rpt)
    return pl.kernel(
        body,
        jax.ShapeDtypeStruct((NC, n_pad, d), jnp.float32),
        mesh=plsc.VectorSubcoreMesh(core_axis_name="c", subcore_axis_name="s"),
        scratch_types=[
            pltpu.VMEM((rpt, CH), jnp.int32),
            pltpu.VMEM((rpt, CH), jnp.int32),
            pltpu.VMEM((CH, d), jnp.float32),
            pltpu.VMEM((16, d), jnp.float32),
            pltpu.VMEM_SHARED((n_pad, d), jnp.float32),
            pltpu.SemaphoreType.DMA,
        ],
    )


def _tc_a_body(deg_ref, x_ref, w_ref, y_ref, dinv_ref):
    dinv = lax.rsqrt(deg_ref[...] + 1.0)
    y_ref[...] = jnp.dot(x_ref[...], w_ref[...],
                         preferred_element_type=jnp.float32) * dinv
    dinv_ref[...] = dinv


def _tc_b_body(p_ref, y1_ref, dinv_ref, b1_ref, y2_ref):
    dinv = dinv_ref[...]
    h = jnp.maximum(dinv * (p_ref[0] + p_ref[1] + y1_ref[...]) + b1_ref[...],
                    0.0)
    y2_ref[...] = h * dinv


def _tc_c_body(q_ref, y2_ref, dinv_ref, b2_ref, w2_ref, out_ref):
    g = dinv_ref[...] * (q_ref[0] + q_ref[1] + y2_ref[...])
    z = jnp.dot(g, w2_ref[...],
                preferred_element_type=jnp.float32) + b2_ref[...]
    m = jnp.max(z, axis=1, keepdims=True)
    lse = jnp.log(jnp.sum(jnp.exp(z - m), axis=1, keepdims=True)) + m
    out_ref[...] = z - lse


def kernel(x, edge_index, W1, b1, W2, b2):
    n, f = x.shape
    nh = W1.shape[1]
    nc = W2.shape[1]
    e = edge_index.shape[1]
    n_pad = ((n + NS * 64 - 1) // (NS * 64)) * (NS * 64)
    e_rows = e // CH
    R = 1000
    grid = (n // R,)

    ei = edge_index.astype(jnp.int32)
    src3 = ei[0].reshape(NC * NS, e_rows // (NC * NS), CH)
    dst3 = ei[1].reshape(NC * NS, e_rows // (NC * NS), CH)
    dst3d = ei[1].reshape(NS, e_rows // NS, CH)

    deg = _make_deg(n_pad, e_rows)(dst3d)
    deg2 = deg.reshape(n_pad, 1)

    y1, dinv = pl.pallas_call(
        _tc_a_body,
        grid=grid,
        in_specs=[
            pl.BlockSpec((R, 1), lambda i: (i, 0)),
            pl.BlockSpec((R, f), lambda i: (i, 0)),
            pl.BlockSpec((f, nh), lambda i: (0, 0)),
        ],
        out_specs=[
            pl.BlockSpec((R, nh), lambda i: (i, 0)),
            pl.BlockSpec((R, 1), lambda i: (i, 0)),
        ],
        out_shape=[
            jax.ShapeDtypeStruct((n, nh), jnp.float32),
            jax.ShapeDtypeStruct((n, 1), jnp.float32),
        ],
    )(deg2, x, W1)

    p = _make_prop(n, nh, n_pad, e_rows)(y1, src3, dst3)

    y2 = pl.pallas_call(
        _tc_b_body,
        grid=grid,
        in_specs=[
            pl.BlockSpec((NC, R, nh), lambda i: (0, i, 0)),
            pl.BlockSpec((R, nh), lambda i: (i, 0)),
            pl.BlockSpec((R, 1), lambda i: (i, 0)),
            pl.BlockSpec((1, nh), lambda i: (0, 0)),
        ],
        out_specs=pl.BlockSpec((R, nh), lambda i: (i, 0)),
        out_shape=jax.ShapeDtypeStruct((n, nh), jnp.float32),
    )(p, y1, dinv, b1.reshape(1, nh))

    q = _make_prop(n, nh, n_pad, e_rows)(y2, src3, dst3)

    out = pl.pallas_call(
        _tc_c_body,
        grid=grid,
        in_specs=[
            pl.BlockSpec((NC, R, nh), lambda i: (0, i, 0)),
            pl.BlockSpec((R, nh), lambda i: (i, 0)),
            pl.BlockSpec((R, 1), lambda i: (i, 0)),
            pl.BlockSpec((1, nc), lambda i: (0, 0)),
            pl.BlockSpec((nh, nc), lambda i: (0, 0)),
        ],
        out_specs=pl.BlockSpec((R, nc), lambda i: (i, 0)),
        out_shape=jax.ShapeDtypeStruct((n, nc), jnp.float32),
    )(q, y2, dinv, b2.reshape(1, nc), W2)

    return out


# speedup vs baseline: 27.7610x; 1.4388x over previous
import functools

import jax
import jax.numpy as jnp
from jax import lax
from jax.experimental import pallas as pl
from jax.experimental.pallas import tpu as pltpu
from jax.experimental.pallas import tpu_sc as plsc

NC = 2
NS = 16
CH = 80


def _deg_body(npt, rows_per_tile, dst_hbm, out_hbm, idx_v, ones_v, zeros_v,
              acc_sh):
    c = lax.axis_index("c")
    s = lax.axis_index("s")

    @pl.when(c == 0)
    def _():
        def fill_ones(i, carry):
            ones_v[pl.ds(i * 16, 16)] = jnp.full((16,), 1.0, jnp.float32)
            return carry
        lax.fori_loop(0, CH // 16, fill_ones, 0)

        def fill_zeros(i, carry):
            zeros_v[pl.ds(i * 16, 16)] = jnp.zeros((16,), jnp.float32)
            return carry
        lax.fori_loop(0, npt // 16, fill_zeros, 0)

        pltpu.sync_copy(zeros_v, acc_sh.at[pl.ds(s * npt, npt)])
        plsc.subcore_barrier()

        pltpu.sync_copy(dst_hbm.at[s], idx_v)

        def chunk(j, carry):
            pltpu.sync_copy(ones_v, acc_sh.at[idx_v.at[j]], add=True)
            return carry
        lax.fori_loop(0, rows_per_tile, chunk, 0)

        plsc.subcore_barrier()
        pltpu.sync_copy(acc_sh.at[pl.ds(s * npt, npt)],
                        out_hbm.at[pl.ds(s * npt, npt)])


def _make_deg(n_pad, e_rows):
    rows_per_tile = e_rows // NS
    npt = n_pad // NS
    body = functools.partial(_deg_body, npt, rows_per_tile)
    return pl.kernel(
        body,
        jax.ShapeDtypeStruct((n_pad,), jnp.float32),
        mesh=plsc.VectorSubcoreMesh(core_axis_name="c", subcore_axis_name="s"),
        scratch_types=[
            pltpu.VMEM((rows_per_tile, CH), jnp.int32),
            pltpu.VMEM((CH,), jnp.float32),
            pltpu.VMEM((npt,), jnp.float32),
            pltpu.VMEM_SHARED((n_pad,), jnp.float32),
        ],
    )


SEG = 5
SEGC = 25


def _prop_body(d, npt, y_hbm, src_hbm, dst_hbm, out_hbm, sidx_v, didx_v,
               rows_a, rows_b, zeros_v, acc_sh, sem_a, sem_b):
    c = lax.axis_index("c")
    s = lax.axis_index("s")
    wid = c * NS + s

    nz = d // 16

    def fill_zeros(i, carry):
        zeros_v[i // nz, pl.ds((i % nz) * 16, 16)] = jnp.zeros((16,),
                                                               jnp.float32)
        return carry
    lax.fori_loop(0, 16 * nz, fill_zeros, 0)

    def zero_acc(j, carry):
        pltpu.sync_copy(zeros_v, acc_sh.at[pl.ds(s * npt + j * 16, 16)])
        return carry
    lax.fori_loop(0, npt // 16, zero_acc, 0)
    plsc.subcore_barrier()

    def wait_g(buf, sem):
        pltpu.make_async_copy(y_hbm.at[pl.ds(0, CH)], buf, sem).wait()

    def seg_loop(g, carry):
        pltpu.sync_copy(src_hbm.at[wid, g], sidx_v)
        pltpu.sync_copy(dst_hbm.at[wid, g], didx_v)
        pltpu.async_copy(y_hbm.at[sidx_v.at[0]], rows_a, sem_a)

        def pair(j, inner):
            c0 = 2 * j
            pltpu.async_copy(y_hbm.at[sidx_v.at[c0 + 1]], rows_b, sem_b)
            wait_g(rows_a, sem_a)
            pltpu.sync_copy(rows_a, acc_sh.at[didx_v.at[c0]], add=True)
            pltpu.async_copy(y_hbm.at[sidx_v.at[c0 + 2]], rows_a, sem_a)
            wait_g(rows_b, sem_b)
            pltpu.sync_copy(rows_b, acc_sh.at[didx_v.at[c0 + 1]], add=True)
            return inner
        lax.fori_loop(0, (SEGC - 1) // 2, pair, 0)

        wait_g(rows_a, sem_a)
        pltpu.sync_copy(rows_a, acc_sh.at[didx_v.at[SEGC - 1]], add=True)
        return carry
    lax.fori_loop(0, SEG, seg_loop, 0)

    plsc.subcore_barrier()
    pltpu.sync_copy(acc_sh.at[pl.ds(s * npt, npt)],
                    out_hbm.at[c, pl.ds(s * npt, npt)])


def _make_prop(n, d, n_pad, e_rows):
    npt = n_pad // NS
    body = functools.partial(_prop_body, d, npt)
    return pl.kernel(
        body,
        jax.ShapeDtypeStruct((NC, n_pad, d), jnp.float32),
        mesh=plsc.VectorSubcoreMesh(core_axis_name="c", subcore_axis_name="s"),
        scratch_types=[
            pltpu.VMEM((SEGC, CH), jnp.int32),
            pltpu.VMEM((SEGC, CH), jnp.int32),
            pltpu.VMEM((CH, d), jnp.float32),
            pltpu.VMEM((CH, d), jnp.float32),
            pltpu.VMEM((16, d), jnp.float32),
            pltpu.VMEM_SHARED((n_pad, d), jnp.float32),
            pltpu.SemaphoreType.DMA,
            pltpu.SemaphoreType.DMA,
        ],
    )


def _tc_a_body(deg_ref, x_ref, w_ref, y_ref, dinv_ref):
    dinv = lax.rsqrt(deg_ref[...] + 1.0)
    y_ref[...] = jnp.dot(x_ref[...], w_ref[...],
                         preferred_element_type=jnp.float32) * dinv
    dinv_ref[...] = dinv


def _tc_b_body(p_ref, y1_ref, dinv_ref, b1_ref, y2_ref):
    dinv = dinv_ref[...]
    h = jnp.maximum(dinv * (p_ref[0] + p_ref[1] + y1_ref[...]) + b1_ref[...],
                    0.0)
    y2_ref[...] = h * dinv


def _tc_c_body(q_ref, y2_ref, dinv_ref, b2_ref, w2_ref, out_ref):
    g = dinv_ref[...] * (q_ref[0] + q_ref[1] + y2_ref[...])
    z = jnp.dot(g, w2_ref[...],
                preferred_element_type=jnp.float32) + b2_ref[...]
    m = jnp.max(z, axis=1, keepdims=True)
    lse = jnp.log(jnp.sum(jnp.exp(z - m), axis=1, keepdims=True)) + m
    out_ref[...] = z - lse


def kernel(x, edge_index, W1, b1, W2, b2):
    n, f = x.shape
    nh = W1.shape[1]
    nc = W2.shape[1]
    e = edge_index.shape[1]
    n_pad = ((n + NS * 64 - 1) // (NS * 64)) * (NS * 64)
    e_rows = e // CH
    R = 1000
    grid = (n // R,)

    ei = edge_index.astype(jnp.int32)
    src3 = ei[0].reshape(NC * NS, SEG, SEGC, CH)
    dst3 = ei[1].reshape(NC * NS, SEG, SEGC, CH)
    dst3d = ei[1].reshape(NS, e_rows // NS, CH)

    deg = _make_deg(n_pad, e_rows)(dst3d)
    deg2 = deg.reshape(n_pad, 1)

    y1, dinv = pl.pallas_call(
        _tc_a_body,
        grid=grid,
        in_specs=[
            pl.BlockSpec((R, 1), lambda i: (i, 0)),
            pl.BlockSpec((R, f), lambda i: (i, 0)),
            pl.BlockSpec((f, nh), lambda i: (0, 0)),
        ],
        out_specs=[
            pl.BlockSpec((R, nh), lambda i: (i, 0)),
            pl.BlockSpec((R, 1), lambda i: (i, 0)),
        ],
        out_shape=[
            jax.ShapeDtypeStruct((n, nh), jnp.float32),
            jax.ShapeDtypeStruct((n, 1), jnp.float32),
        ],
    )(deg2, x, W1)

    p = _make_prop(n, nh, n_pad, e_rows)(y1, src3, dst3)

    y2 = pl.pallas_call(
        _tc_b_body,
        grid=grid,
        in_specs=[
            pl.BlockSpec((NC, R, nh), lambda i: (0, i, 0)),
            pl.BlockSpec((R, nh), lambda i: (i, 0)),
            pl.BlockSpec((R, 1), lambda i: (i, 0)),
            pl.BlockSpec((1, nh), lambda i: (0, 0)),
        ],
        out_specs=pl.BlockSpec((R, nh), lambda i: (i, 0)),
        out_shape=jax.ShapeDtypeStruct((n, nh), jnp.float32),
    )(p, y1, dinv, b1.reshape(1, nh))

    q = _make_prop(n, nh, n_pad, e_rows)(y2, src3, dst3)

    out = pl.pallas_call(
        _tc_c_body,
        grid=grid,
        in_specs=[
            pl.BlockSpec((NC, R, nh), lambda i: (0, i, 0)),
            pl.BlockSpec((R, nh), lambda i: (i, 0)),
            pl.BlockSpec((R, 1), lambda i: (i, 0)),
            pl.BlockSpec((1, nc), lambda i: (0, 0)),
            pl.BlockSpec((nh, nc), lambda i: (0, 0)),
        ],
        out_specs=pl.BlockSpec((R, nc), lambda i: (i, 0)),
        out_shape=jax.ShapeDtypeStruct((n, nc), jnp.float32),
    )(q, y2, dinv, b2.reshape(1, nc), W2)

    return out


# speedup vs baseline: 27.8663x; 1.0038x over previous
import functools

import jax
import jax.numpy as jnp
from jax import lax
from jax.experimental import pallas as pl
from jax.experimental.pallas import tpu as pltpu
from jax.experimental.pallas import tpu_sc as plsc

NC = 2
NS = 16
CH = 80
SEG = 5
SEGC = 25


def _deg_body(npt, dst_hbm, out0_hbm, out1_hbm, didx_v, ones_v, zeros_v,
              acc_sh):
    c = lax.axis_index("c")
    s = lax.axis_index("s")
    wid = c * NS + s

    def fill_ones(i, carry):
        ones_v[pl.ds(i * 16, 16)] = jnp.full((16,), 1.0, jnp.float32)
        return carry
    lax.fori_loop(0, CH // 16, fill_ones, 0)

    def fill_zeros(i, carry):
        zeros_v[pl.ds(i * 16, 16)] = jnp.zeros((16,), jnp.float32)
        return carry
    lax.fori_loop(0, 640 // 16, fill_zeros, 0)

    pltpu.sync_copy(zeros_v.at[pl.ds(0, npt)], acc_sh.at[pl.ds(s * npt, npt)])
    plsc.subcore_barrier()

    def seg_loop(g, carry):
        pltpu.sync_copy(dst_hbm.at[wid, g], didx_v)

        def fire(j, inner):
            pltpu.sync_copy(ones_v, acc_sh.at[didx_v.at[j]], add=True)
            return inner
        lax.fori_loop(0, SEGC, fire, 0)
        return carry
    lax.fori_loop(0, SEG, seg_loop, 0)

    plsc.subcore_barrier()

    @pl.when(c == 0)
    def _():
        pltpu.sync_copy(acc_sh.at[pl.ds(s * npt, npt)],
                        out0_hbm.at[pl.ds(s * npt, npt)])

    @pl.when(c == 1)
    def _():
        pltpu.sync_copy(acc_sh.at[pl.ds(s * npt, npt)],
                        out1_hbm.at[pl.ds(s * npt, npt)])


def _make_deg(n_pad):
    npt = n_pad // NS
    body = functools.partial(_deg_body, npt)
    return pl.kernel(
        body,
        [jax.ShapeDtypeStruct((n_pad,), jnp.float32),
         jax.ShapeDtypeStruct((n_pad,), jnp.float32)],
        mesh=plsc.VectorSubcoreMesh(core_axis_name="c", subcore_axis_name="s"),
        scratch_types=[
            pltpu.VMEM((SEGC, CH), jnp.int32),
            pltpu.VMEM((CH,), jnp.float32),
            pltpu.VMEM((640,), jnp.float32),
            pltpu.VMEM_SHARED((n_pad,), jnp.float32),
        ],
    )


def _prop_body(d, npt, y_hbm, src_hbm, dst_hbm, out_hbm, sidx_v, didx_v,
               rows_a, rows_b, zeros_v, acc_sh, sem_ga, sem_gb):
    c = lax.axis_index("c")
    s = lax.axis_index("s")
    wid = c * NS + s

    nz = d // 16

    def fill_zeros(i, carry):
        zeros_v[i // nz, pl.ds((i % nz) * 16, 16)] = jnp.zeros((16,),
                                                               jnp.float32)
        return carry
    lax.fori_loop(0, 8 * nz, fill_zeros, 0)

    def zero_acc(j, carry):
        pltpu.sync_copy(zeros_v, acc_sh.at[pl.ds(s * npt + j * 8, 8)])
        return carry
    lax.fori_loop(0, npt // 8, zero_acc, 0)
    plsc.subcore_barrier()

    def g_start(buf, sem, chunk):
        pltpu.async_copy(y_hbm.at[sidx_v.at[chunk]], buf, sem)

    def g_wait(buf, sem):
        pltpu.make_async_copy(y_hbm.at[pl.ds(0, CH)], buf, sem).wait()

    def seg_loop(g, carry):
        pltpu.sync_copy(src_hbm.at[wid, g], sidx_v)
        pltpu.sync_copy(dst_hbm.at[wid, g], didx_v)
        g_start(rows_a, sem_ga, 0)

        def pair(j, inner):
            c0 = 2 * j
            g_start(rows_b, sem_gb, c0 + 1)
            g_wait(rows_a, sem_ga)
            pltpu.sync_copy(rows_a, acc_sh.at[didx_v.at[c0]], add=True)
            g_start(rows_a, sem_ga, c0 + 2)
            g_wait(rows_b, sem_gb)
            pltpu.sync_copy(rows_b, acc_sh.at[didx_v.at[c0 + 1]], add=True)
            return inner
        lax.fori_loop(0, (SEGC - 1) // 2, pair, 0)

        g_wait(rows_a, sem_ga)
        pltpu.sync_copy(rows_a, acc_sh.at[didx_v.at[SEGC - 1]], add=True)
        return carry
    lax.fori_loop(0, SEG, seg_loop, 0)

    plsc.subcore_barrier()
    pltpu.sync_copy(acc_sh.at[pl.ds(s * npt, npt)],
                    out_hbm.at[c, pl.ds(s * npt, npt)])


def _make_prop(n, d, n_pad):
    npt = n_pad // NS
    body = functools.partial(_prop_body, d, npt)
    return pl.kernel(
        body,
        jax.ShapeDtypeStruct((NC, n_pad, d), jnp.float32),
        mesh=plsc.VectorSubcoreMesh(core_axis_name="c", subcore_axis_name="s"),
        scratch_types=[
            pltpu.VMEM((SEGC, CH), jnp.int32),
            pltpu.VMEM((SEGC, CH), jnp.int32),
            pltpu.VMEM((CH, d), jnp.float32),
            pltpu.VMEM((CH, d), jnp.float32),
            pltpu.VMEM((8, d), jnp.float32),
            pltpu.VMEM_SHARED((n_pad, d), jnp.float32),
            pltpu.SemaphoreType.DMA,
            pltpu.SemaphoreType.DMA,
        ],
    )


def _tc_a_body(deg0_ref, deg1_ref, x_ref, w_ref, y_ref, dinv_ref):
    dinv = lax.rsqrt(deg0_ref[...] + deg1_ref[...] + 1.0)
    y_ref[...] = jnp.dot(x_ref[...], w_ref[...],
                         preferred_element_type=jnp.float32) * dinv
    dinv_ref[...] = dinv


def _tc_b_body(p_ref, y1_ref, dinv_ref, b1_ref, y2_ref):
    dinv = dinv_ref[...]
    h = jnp.maximum(dinv * (p_ref[0] + p_ref[1] + y1_ref[...]) + b1_ref[...],
                    0.0)
    y2_ref[...] = h * dinv


def _tc_c_body(q_ref, y2_ref, dinv_ref, b2_ref, w2_ref, out_ref):
    g = dinv_ref[...] * (q_ref[0] + q_ref[1] + y2_ref[...])
    z = jnp.dot(g, w2_ref[...],
                preferred_element_type=jnp.float32) + b2_ref[...]
    m = jnp.max(z, axis=1, keepdims=True)
    lse = jnp.log(jnp.sum(jnp.exp(z - m), axis=1, keepdims=True)) + m
    out_ref[...] = z - lse


def kernel(x, edge_index, W1, b1, W2, b2):
    n, f = x.shape
    nh = W1.shape[1]
    nc = W2.shape[1]
    e = edge_index.shape[1]
    n_pad = ((n + NS * 16 - 1) // (NS * 16)) * (NS * 16)
    R = 1000
    grid = (n // R,)

    ei = edge_index.astype(jnp.int32)
    src4 = ei[0].reshape(NC * NS, SEG, SEGC, CH)
    dst4 = ei[1].reshape(NC * NS, SEG, SEGC, CH)

    deg0, deg1 = _make_deg(n_pad)(dst4)

    y1, dinv = pl.pallas_call(
        _tc_a_body,
        grid=grid,
        in_specs=[
            pl.BlockSpec((R, 1), lambda i: (i, 0)),
            pl.BlockSpec((R, 1), lambda i: (i, 0)),
            pl.BlockSpec((R, f), lambda i: (i, 0)),
            pl.BlockSpec((f, nh), lambda i: (0, 0)),
        ],
        out_specs=[
            pl.BlockSpec((R, nh), lambda i: (i, 0)),
            pl.BlockSpec((R, 1), lambda i: (i, 0)),
        ],
        out_shape=[
            jax.ShapeDtypeStruct((n, nh), jnp.float32),
            jax.ShapeDtypeStruct((n, 1), jnp.float32),
        ],
    )(deg0.reshape(n_pad, 1), deg1.reshape(n_pad, 1), x, W1)

    p = _make_prop(n, nh, n_pad)(y1, src4, dst4)

    y2 = pl.pallas_call(
        _tc_b_body,
        grid=grid,
        in_specs=[
            pl.BlockSpec((NC, R, nh), lambda i: (0, i, 0)),
            pl.BlockSpec((R, nh), lambda i: (i, 0)),
            pl.BlockSpec((R, 1), lambda i: (i, 0)),
            pl.BlockSpec((1, nh), lambda i: (0, 0)),
        ],
        out_specs=pl.BlockSpec((R, nh), lambda i: (i, 0)),
        out_shape=jax.ShapeDtypeStruct((n, nh), jnp.float32),
    )(p, y1, dinv, b1.reshape(1, nh))

    q = _make_prop(n, nh, n_pad)(y2, src4, dst4)

    out = pl.pallas_call(
        _tc_c_body,
        grid=grid,
        in_specs=[
            pl.BlockSpec((NC, R, nh), lambda i: (0, i, 0)),
            pl.BlockSpec((R, nh), lambda i: (i, 0)),
            pl.BlockSpec((R, 1), lambda i: (i, 0)),
            pl.BlockSpec((1, nc), lambda i: (0, 0)),
            pl.BlockSpec((nh, nc), lambda i: (0, 0)),
        ],
        out_specs=pl.BlockSpec((R, nc), lambda i: (i, 0)),
        out_shape=jax.ShapeDtypeStruct((n, nc), jnp.float32),
    )(q, y2, dinv, b2.reshape(1, nc), W2)

    return out


# speedup vs baseline: 30.8401x; 1.1067x over previous
import functools

import jax
import jax.numpy as jnp
from jax import lax
from jax.experimental import pallas as pl
from jax.experimental.pallas import tpu as pltpu
from jax.experimental.pallas import tpu_sc as plsc

NC = 2
NS = 16
CH = 80
SEG = 5
SEGC = 25


def _deg_body(npt, dst_hbm, out0_hbm, out1_hbm, didx_v, ones_v, zeros_v,
              acc_sh):
    c = lax.axis_index("c")
    s = lax.axis_index("s")
    wid = c * NS + s

    def fill_ones(i, carry):
        ones_v[pl.ds(i * 16, 16)] = jnp.full((16,), 1.0, jnp.float32)
        return carry
    lax.fori_loop(0, CH // 16, fill_ones, 0)

    def fill_zeros(i, carry):
        zeros_v[pl.ds(i * 16, 16)] = jnp.zeros((16,), jnp.float32)
        return carry
    lax.fori_loop(0, 640 // 16, fill_zeros, 0)

    pltpu.sync_copy(zeros_v.at[pl.ds(0, npt)], acc_sh.at[pl.ds(s * npt, npt)])
    plsc.subcore_barrier()

    def seg_loop(g, carry):
        pltpu.sync_copy(dst_hbm.at[wid, g], didx_v)

        def fire(j, inner):
            pltpu.sync_copy(ones_v, acc_sh.at[didx_v.at[j]], add=True)
            return inner
        lax.fori_loop(0, SEGC, fire, 0)
        return carry
    lax.fori_loop(0, SEG, seg_loop, 0)

    plsc.subcore_barrier()

    @pl.when(c == 0)
    def _():
        pltpu.sync_copy(acc_sh.at[pl.ds(s * npt, npt)],
                        out0_hbm.at[pl.ds(s * npt, npt)])

    @pl.when(c == 1)
    def _():
        pltpu.sync_copy(acc_sh.at[pl.ds(s * npt, npt)],
                        out1_hbm.at[pl.ds(s * npt, npt)])


def _make_deg(n_pad):
    npt = n_pad // NS
    body = functools.partial(_deg_body, npt)
    return pl.kernel(
        body,
        [jax.ShapeDtypeStruct((n_pad,), jnp.float32),
         jax.ShapeDtypeStruct((n_pad,), jnp.float32)],
        mesh=plsc.VectorSubcoreMesh(core_axis_name="c", subcore_axis_name="s"),
        scratch_types=[
            pltpu.VMEM((SEGC, CH), jnp.int32),
            pltpu.VMEM((CH,), jnp.float32),
            pltpu.VMEM((640,), jnp.float32),
            pltpu.VMEM_SHARED((n_pad,), jnp.float32),
        ],
    )


def _prop_body(d, npt, y_hbm, src_hbm, dst_hbm, out_hbm, sidx_v, didx_v,
               rows_a, rows_b, rows_c, zeros_v, acc_sh,
               sem_ga, sem_gb, sem_gc, sem_sa, sem_sb, sem_sc):
    c = lax.axis_index("c")
    s = lax.axis_index("s")
    wid = c * NS + s

    nz = d // 16

    def fill_zeros(i, carry):
        zeros_v[i // nz, pl.ds((i % nz) * 16, 16)] = jnp.zeros((16,),
                                                               jnp.float32)
        return carry
    lax.fori_loop(0, 8 * nz, fill_zeros, 0)

    def zero_acc(j, carry):
        pltpu.sync_copy(zeros_v, acc_sh.at[pl.ds(s * npt + j * 8, 8)])
        return carry
    lax.fori_loop(0, npt // 8, zero_acc, 0)
    plsc.subcore_barrier()

    def g_start(buf, sem, chunk):
        pltpu.async_copy(y_hbm.at[sidx_v.at[chunk]], buf, sem)

    def g_wait(buf, sem):
        pltpu.make_async_copy(y_hbm.at[pl.ds(0, CH)], buf, sem).wait()

    def s_start(buf, sem, chunk):
        pltpu.async_copy(buf, acc_sh.at[didx_v.at[chunk]], sem, add=True)

    def s_wait(buf, sem, chunk):
        pltpu.make_async_copy(buf, acc_sh.at[didx_v.at[chunk]], sem).wait()

    def seg_loop(g, carry):
        pltpu.sync_copy(src_hbm.at[wid, g], sidx_v)
        pltpu.sync_copy(dst_hbm.at[wid, g], didx_v)
        g_start(rows_a, sem_ga, 0)
        g_start(rows_b, sem_gb, 1)

        def triple(j, inner):
            c0 = 3 * j
            g_wait(rows_a, sem_ga)
            s_start(rows_a, sem_sa, c0)

            @pl.when(j > 0)
            def _():
                s_wait(rows_c, sem_sc, c0 - 1)
            g_start(rows_c, sem_gc, c0 + 2)

            g_wait(rows_b, sem_gb)
            s_start(rows_b, sem_sb, c0 + 1)

            s_wait(rows_a, sem_sa, c0)
            g_start(rows_a, sem_ga, c0 + 3)

            g_wait(rows_c, sem_gc)
            s_start(rows_c, sem_sc, c0 + 2)

            s_wait(rows_b, sem_sb, c0 + 1)

            @pl.when(j < (SEGC - 1) // 3 - 1)
            def _():
                g_start(rows_b, sem_gb, c0 + 4)
            return inner
        lax.fori_loop(0, (SEGC - 1) // 3, triple, 0)

        g_wait(rows_a, sem_ga)
        s_start(rows_a, sem_sa, SEGC - 1)
        s_wait(rows_c, sem_sc, SEGC - 2)
        s_wait(rows_a, sem_sa, SEGC - 1)
        return carry
    lax.fori_loop(0, SEG, seg_loop, 0)

    plsc.subcore_barrier()
    pltpu.sync_copy(acc_sh.at[pl.ds(s * npt, npt)],
                    out_hbm.at[c, pl.ds(s * npt, npt)])


def _make_prop(n, d, n_pad):
    npt = n_pad // NS
    body = functools.partial(_prop_body, d, npt)
    return pl.kernel(
        body,
        jax.ShapeDtypeStruct((NC, n_pad, d), jnp.float32),
        mesh=plsc.VectorSubcoreMesh(core_axis_name="c", subcore_axis_name="s"),
        scratch_types=[
            pltpu.VMEM((SEGC, CH), jnp.int32),
            pltpu.VMEM((SEGC, CH), jnp.int32),
            pltpu.VMEM((CH, d), jnp.float32),
            pltpu.VMEM((CH, d), jnp.float32),
            pltpu.VMEM((CH, d), jnp.float32),
            pltpu.VMEM((8, d), jnp.float32),
            pltpu.VMEM_SHARED((n_pad, d), jnp.float32),
            pltpu.SemaphoreType.DMA,
            pltpu.SemaphoreType.DMA,
            pltpu.SemaphoreType.DMA,
            pltpu.SemaphoreType.DMA,
            pltpu.SemaphoreType.DMA,
            pltpu.SemaphoreType.DMA,
        ],
    )


def _tc_a_body(deg0_ref, deg1_ref, x_ref, w_ref, y_ref, dinv_ref):
    dinv = lax.rsqrt(deg0_ref[...] + deg1_ref[...] + 1.0)
    y_ref[...] = jnp.dot(x_ref[...], w_ref[...],
                         preferred_element_type=jnp.float32) * dinv
    dinv_ref[...] = dinv


def _tc_b_body(p_ref, y1_ref, dinv_ref, b1_ref, y2_ref):
    dinv = dinv_ref[...]
    h = jnp.maximum(dinv * (p_ref[0] + p_ref[1] + y1_ref[...]) + b1_ref[...],
                    0.0)
    y2_ref[...] = h * dinv


def _tc_c_body(q_ref, y2_ref, dinv_ref, b2_ref, w2_ref, out_ref):
    g = dinv_ref[...] * (q_ref[0] + q_ref[1] + y2_ref[...])
    z = jnp.dot(g, w2_ref[...],
                preferred_element_type=jnp.float32) + b2_ref[...]
    m = jnp.max(z, axis=1, keepdims=True)
    lse = jnp.log(jnp.sum(jnp.exp(z - m), axis=1, keepdims=True)) + m
    out_ref[...] = z - lse


def kernel(x, edge_index, W1, b1, W2, b2):
    n, f = x.shape
    nh = W1.shape[1]
    nc = W2.shape[1]
    e = edge_index.shape[1]
    n_pad = ((n + NS * 16 - 1) // (NS * 16)) * (NS * 16)
    R = 1000
    grid = (n // R,)

    ei = edge_index.astype(jnp.int32)
    src4 = ei[0].reshape(NC * NS, SEG, SEGC, CH)
    dst4 = ei[1].reshape(NC * NS, SEG, SEGC, CH)

    deg0, deg1 = _make_deg(n_pad)(dst4)

    y1, dinv = pl.pallas_call(
        _tc_a_body,
        grid=grid,
        in_specs=[
            pl.BlockSpec((R, 1), lambda i: (i, 0)),
            pl.BlockSpec((R, 1), lambda i: (i, 0)),
            pl.BlockSpec((R, f), lambda i: (i, 0)),
            pl.BlockSpec((f, nh), lambda i: (0, 0)),
        ],
        out_specs=[
            pl.BlockSpec((R, nh), lambda i: (i, 0)),
            pl.BlockSpec((R, 1), lambda i: (i, 0)),
        ],
        out_shape=[
            jax.ShapeDtypeStruct((n, nh), jnp.float32),
            jax.ShapeDtypeStruct((n, 1), jnp.float32),
        ],
    )(deg0.reshape(n_pad, 1), deg1.reshape(n_pad, 1), x, W1)

    p = _make_prop(n, nh, n_pad)(y1, src4, dst4)

    y2 = pl.pallas_call(
        _tc_b_body,
        grid=grid,
        in_specs=[
            pl.BlockSpec((NC, R, nh), lambda i: (0, i, 0)),
            pl.BlockSpec((R, nh), lambda i: (i, 0)),
            pl.BlockSpec((R, 1), lambda i: (i, 0)),
            pl.BlockSpec((1, nh), lambda i: (0, 0)),
        ],
        out_specs=pl.BlockSpec((R, nh), lambda i: (i, 0)),
        out_shape=jax.ShapeDtypeStruct((n, nh), jnp.float32),
    )(p, y1, dinv, b1.reshape(1, nh))

    q = _make_prop(n, nh, n_pad)(y2, src4, dst4)

    out = pl.pallas_call(
        _tc_c_body,
        grid=grid,
        in_specs=[
            pl.BlockSpec((NC, R, nh), lambda i: (0, i, 0)),
            pl.BlockSpec((R, nh), lambda i: (i, 0)),
            pl.BlockSpec((R, 1), lambda i: (i, 0)),
            pl.BlockSpec((1, nc), lambda i: (0, 0)),
            pl.BlockSpec((nh, nc), lambda i: (0, 0)),
        ],
        out_specs=pl.BlockSpec((R, nc), lambda i: (i, 0)),
        out_shape=jax.ShapeDtypeStruct((n, nc), jnp.float32),
    )(q, y2, dinv, b2.reshape(1, nc), W2)

    return out


# speedup vs baseline: 31.0264x; 1.0060x over previous
import functools

import jax
import jax.numpy as jnp
from jax import lax
from jax.experimental import pallas as pl
from jax.experimental.pallas import tpu as pltpu
from jax.experimental.pallas import tpu_sc as plsc

NC = 2
NS = 16
CH = 80
SEG = 5
SEGC = 25


def _deg_body(npt, edge_hbm, out0_hbm, out1_hbm, didx_v, ones_v, zeros_v,
              acc_sh):
    c = lax.axis_index("c")
    s = lax.axis_index("s")
    wid = c * NS + s

    def fill_ones(i, carry):
        ones_v[pl.ds(i * 16, 16)] = jnp.full((16,), 1.0, jnp.float32)
        return carry
    lax.fori_loop(0, CH // 16, fill_ones, 0)

    def fill_zeros(i, carry):
        zeros_v[pl.ds(i * 16, 16)] = jnp.zeros((16,), jnp.float32)
        return carry
    lax.fori_loop(0, 640 // 16, fill_zeros, 0)

    pltpu.sync_copy(zeros_v.at[pl.ds(0, npt)], acc_sh.at[pl.ds(s * npt, npt)])
    plsc.subcore_barrier()

    def seg_loop(g, carry):
        pltpu.sync_copy(edge_hbm.at[wid, g, 1], didx_v)

        def fire(j, inner):
            pltpu.sync_copy(ones_v, acc_sh.at[didx_v.at[j]], add=True)
            return inner
        lax.fori_loop(0, SEGC, fire, 0)
        return carry
    lax.fori_loop(0, SEG, seg_loop, 0)

    plsc.subcore_barrier()

    @pl.when(c == 0)
    def _():
        pltpu.sync_copy(acc_sh.at[pl.ds(s * npt, npt)],
                        out0_hbm.at[pl.ds(s * npt, npt)])

    @pl.when(c == 1)
    def _():
        pltpu.sync_copy(acc_sh.at[pl.ds(s * npt, npt)],
                        out1_hbm.at[pl.ds(s * npt, npt)])


def _make_deg(n_pad):
    npt = n_pad // NS
    body = functools.partial(_deg_body, npt)
    return pl.kernel(
        body,
        [jax.ShapeDtypeStruct((n_pad,), jnp.float32),
         jax.ShapeDtypeStruct((n_pad,), jnp.float32)],
        mesh=plsc.VectorSubcoreMesh(core_axis_name="c", subcore_axis_name="s"),
        scratch_types=[
            pltpu.VMEM((SEGC, CH), jnp.int32),
            pltpu.VMEM((CH,), jnp.float32),
            pltpu.VMEM((640,), jnp.float32),
            pltpu.VMEM_SHARED((n_pad,), jnp.float32),
        ],
    )


def _prop_body(d, npt, y_hbm, edge_hbm, out_hbm, idx_v,
               rows_a, rows_b, rows_c, zeros_v, acc_sh,
               sem_ga, sem_gb, sem_gc, sem_sa, sem_sb, sem_sc):
    c = lax.axis_index("c")
    s = lax.axis_index("s")
    wid = c * NS + s

    nz = d // 16

    def fill_zeros(i, carry):
        zeros_v[i // nz, pl.ds((i % nz) * 16, 16)] = jnp.zeros((16,),
                                                               jnp.float32)
        return carry
    lax.fori_loop(0, 8 * nz, fill_zeros, 0)

    def zero_acc(j, carry):
        pltpu.sync_copy(zeros_v, acc_sh.at[pl.ds(s * npt + j * 8, 8)])
        return carry
    lax.fori_loop(0, npt // 8, zero_acc, 0)
    plsc.subcore_barrier()

    def g_start(buf, sem, chunk):
        pltpu.async_copy(y_hbm.at[idx_v.at[0, chunk]], buf, sem)

    def g_wait(buf, sem):
        pltpu.make_async_copy(y_hbm.at[pl.ds(0, CH)], buf, sem).wait()

    def s_start(buf, sem, chunk):
        pltpu.async_copy(buf, acc_sh.at[idx_v.at[1, chunk]], sem, add=True)

    def s_wait(buf, sem, chunk):
        pltpu.make_async_copy(buf, acc_sh.at[idx_v.at[1, chunk]], sem).wait()

    def seg_loop(g, carry):
        pltpu.sync_copy(edge_hbm.at[wid, g], idx_v)
        g_start(rows_a, sem_ga, 0)
        g_start(rows_b, sem_gb, 1)

        def triple(j, inner):
            c0 = 3 * j
            g_wait(rows_a, sem_ga)
            s_start(rows_a, sem_sa, c0)

            @pl.when(j > 0)
            def _():
                s_wait(rows_c, sem_sc, c0 - 1)
            g_start(rows_c, sem_gc, c0 + 2)

            g_wait(rows_b, sem_gb)
            s_start(rows_b, sem_sb, c0 + 1)

            s_wait(rows_a, sem_sa, c0)
            g_start(rows_a, sem_ga, c0 + 3)

            g_wait(rows_c, sem_gc)
            s_start(rows_c, sem_sc, c0 + 2)

            s_wait(rows_b, sem_sb, c0 + 1)

            @pl.when(j < (SEGC - 1) // 3 - 1)
            def _():
                g_start(rows_b, sem_gb, c0 + 4)
            return inner
        lax.fori_loop(0, (SEGC - 1) // 3, triple, 0)

        g_wait(rows_a, sem_ga)
        s_start(rows_a, sem_sa, SEGC - 1)
        s_wait(rows_c, sem_sc, SEGC - 2)
        s_wait(rows_a, sem_sa, SEGC - 1)
        return carry
    lax.fori_loop(0, SEG, seg_loop, 0)

    plsc.subcore_barrier()
    pltpu.sync_copy(acc_sh.at[pl.ds(s * npt, npt)],
                    out_hbm.at[c, pl.ds(s * npt, npt)])


def _make_prop(n, d, n_pad):
    npt = n_pad // NS
    body = functools.partial(_prop_body, d, npt)
    return pl.kernel(
        body,
        jax.ShapeDtypeStruct((NC, n_pad, d), jnp.float32),
        mesh=plsc.VectorSubcoreMesh(core_axis_name="c", subcore_axis_name="s"),
        scratch_types=[
            pltpu.VMEM((2, SEGC, CH), jnp.int32),
            pltpu.VMEM((CH, d), jnp.float32),
            pltpu.VMEM((CH, d), jnp.float32),
            pltpu.VMEM((CH, d), jnp.float32),
            pltpu.VMEM((8, d), jnp.float32),
            pltpu.VMEM_SHARED((n_pad, d), jnp.float32),
            pltpu.SemaphoreType.DMA,
            pltpu.SemaphoreType.DMA,
            pltpu.SemaphoreType.DMA,
            pltpu.SemaphoreType.DMA,
            pltpu.SemaphoreType.DMA,
            pltpu.SemaphoreType.DMA,
        ],
    )


def _tc_a1_body(x_ref, w_ref, xw_ref):
    xw_ref[...] = jnp.dot(x_ref[...], w_ref[...],
                          preferred_element_type=jnp.float32)


def _tc_a2_body(deg0_ref, deg1_ref, xw_ref, y_ref, dinv_ref):
    dinv = lax.rsqrt(deg0_ref[...] + deg1_ref[...] + 1.0)
    y_ref[...] = xw_ref[...] * dinv
    dinv_ref[...] = dinv


def _tc_b_body(p_ref, y1_ref, dinv_ref, b1_ref, y2_ref):
    dinv = dinv_ref[...]
    h = jnp.maximum(dinv * (p_ref[0] + p_ref[1] + y1_ref[...]) + b1_ref[...],
                    0.0)
    y2_ref[...] = h * dinv


def _tc_c_body(q_ref, y2_ref, dinv_ref, b2_ref, w2_ref, out_ref):
    g = dinv_ref[...] * (q_ref[0] + q_ref[1] + y2_ref[...])
    z = jnp.dot(g, w2_ref[...],
                preferred_element_type=jnp.float32) + b2_ref[...]
    m = jnp.max(z, axis=1, keepdims=True)
    lse = jnp.log(jnp.sum(jnp.exp(z - m), axis=1, keepdims=True)) + m
    out_ref[...] = z - lse


def kernel(x, edge_index, W1, b1, W2, b2):
    n, f = x.shape
    nh = W1.shape[1]
    nc = W2.shape[1]
    e = edge_index.shape[1]
    n_pad = ((n + NS * 16 - 1) // (NS * 16)) * (NS * 16)
    R = 1000
    grid = (n // R,)

    ei = edge_index.astype(jnp.int32)
    edge5 = jnp.stack(
        [ei[0].reshape(NC * NS, SEG, SEGC, CH),
         ei[1].reshape(NC * NS, SEG, SEGC, CH)], axis=2)

    deg0, deg1 = _make_deg(n_pad)(edge5)

    xw = pl.pallas_call(
        _tc_a1_body,
        grid=grid,
        in_specs=[
            pl.BlockSpec((R, f), lambda i: (i, 0)),
            pl.BlockSpec((f, nh), lambda i: (0, 0)),
        ],
        out_specs=pl.BlockSpec((R, nh), lambda i: (i, 0)),
        out_shape=jax.ShapeDtypeStruct((n, nh), jnp.float32),
    )(x, W1)

    y1, dinv = pl.pallas_call(
        _tc_a2_body,
        grid=grid,
        in_specs=[
            pl.BlockSpec((R, 1), lambda i: (i, 0)),
            pl.BlockSpec((R, 1), lambda i: (i, 0)),
            pl.BlockSpec((R, nh), lambda i: (i, 0)),
        ],
        out_specs=[
            pl.BlockSpec((R, nh), lambda i: (i, 0)),
            pl.BlockSpec((R, 1), lambda i: (i, 0)),
        ],
        out_shape=[
            jax.ShapeDtypeStruct((n, nh), jnp.float32),
            jax.ShapeDtypeStruct((n, 1), jnp.float32),
        ],
    )(deg0.reshape(n_pad, 1), deg1.reshape(n_pad, 1), xw)

    p = _make_prop(n, nh, n_pad)(y1, edge5)

    y2 = pl.pallas_call(
        _tc_b_body,
        grid=grid,
        in_specs=[
            pl.BlockSpec((NC, R, nh), lambda i: (0, i, 0)),
            pl.BlockSpec((R, nh), lambda i: (i, 0)),
            pl.BlockSpec((R, 1), lambda i: (i, 0)),
            pl.BlockSpec((1, nh), lambda i: (0, 0)),
        ],
        out_specs=pl.BlockSpec((R, nh), lambda i: (i, 0)),
        out_shape=jax.ShapeDtypeStruct((n, nh), jnp.float32),
    )(p, y1, dinv, b1.reshape(1, nh))

    q = _make_prop(n, nh, n_pad)(y2, edge5)

    out = pl.pallas_call(
        _tc_c_body,
        grid=grid,
        in_specs=[
            pl.BlockSpec((NC, R, nh), lambda i: (0, i, 0)),
            pl.BlockSpec((R, nh), lambda i: (i, 0)),
            pl.BlockSpec((R, 1), lambda i: (i, 0)),
            pl.BlockSpec((1, nc), lambda i: (0, 0)),
            pl.BlockSpec((nh, nc), lambda i: (0, 0)),
        ],
        out_specs=pl.BlockSpec((R, nc), lambda i: (i, 0)),
        out_shape=jax.ShapeDtypeStruct((n, nc), jnp.float32),
    )(q, y2, dinv, b2.reshape(1, nc), W2)

    return out


# speedup vs baseline: 32.1054x; 1.0348x over previous
import functools

import jax
import jax.numpy as jnp
from jax import lax
from jax.experimental import pallas as pl
from jax.experimental.pallas import tpu as pltpu
from jax.experimental.pallas import tpu_sc as plsc

NC = 2
NS = 16
CH = 80
SEG = 5
SEGC = 25


def _deg_body(npt, dst_hbm, out0_hbm, out1_hbm, didx_v, ones_v, zeros_v,
              acc_sh):
    c = lax.axis_index("c")
    s = lax.axis_index("s")
    wid = c * NS + s

    def fill_ones(i, carry):
        ones_v[pl.ds(i * 16, 16)] = jnp.full((16,), 1.0, jnp.float32)
        return carry
    lax.fori_loop(0, CH // 16, fill_ones, 0)

    def fill_zeros(i, carry):
        zeros_v[pl.ds(i * 16, 16)] = jnp.zeros((16,), jnp.float32)
        return carry
    lax.fori_loop(0, 640 // 16, fill_zeros, 0)

    pltpu.sync_copy(zeros_v.at[pl.ds(0, npt)], acc_sh.at[pl.ds(s * npt, npt)])
    plsc.subcore_barrier()

    def seg_loop(g, carry):
        pltpu.sync_copy(dst_hbm.at[wid, g], didx_v)

        def fire(j, inner):
            pltpu.sync_copy(ones_v, acc_sh.at[didx_v.at[j]], add=True)
            return inner
        lax.fori_loop(0, SEGC, fire, 0)
        return carry
    lax.fori_loop(0, SEG, seg_loop, 0)

    plsc.subcore_barrier()

    @pl.when(c == 0)
    def _():
        pltpu.sync_copy(acc_sh.at[pl.ds(s * npt, npt)],
                        out0_hbm.at[pl.ds(s * npt, npt)])

    @pl.when(c == 1)
    def _():
        pltpu.sync_copy(acc_sh.at[pl.ds(s * npt, npt)],
                        out1_hbm.at[pl.ds(s * npt, npt)])


def _make_deg(n_pad):
    npt = n_pad // NS
    body = functools.partial(_deg_body, npt)
    return pl.kernel(
        body,
        [jax.ShapeDtypeStruct((n_pad,), jnp.float32),
         jax.ShapeDtypeStruct((n_pad,), jnp.float32)],
        mesh=plsc.VectorSubcoreMesh(core_axis_name="c", subcore_axis_name="s"),
        scratch_types=[
            pltpu.VMEM((SEGC, CH), jnp.int32),
            pltpu.VMEM((CH,), jnp.float32),
            pltpu.VMEM((640,), jnp.float32),
            pltpu.VMEM_SHARED((n_pad,), jnp.float32),
        ],
    )


def _prop_body(d, npt, y_hbm, src_hbm, dst_hbm, out_hbm, idx_v,
               rows_a, rows_b, rows_c, zeros_v, acc_sh,
               sem_ga, sem_gb, sem_gc, sem_sa, sem_sb, sem_sc):
    c = lax.axis_index("c")
    s = lax.axis_index("s")
    wid = c * NS + s

    nz = d // 16

    def fill_zeros(i, carry):
        zeros_v[i // nz, pl.ds((i % nz) * 16, 16)] = jnp.zeros((16,),
                                                               jnp.float32)
        return carry
    lax.fori_loop(0, 8 * nz, fill_zeros, 0)

    def zero_acc(j, carry):
        pltpu.sync_copy(zeros_v, acc_sh.at[pl.ds(s * npt + j * 8, 8)])
        return carry
    lax.fori_loop(0, npt // 8, zero_acc, 0)
    plsc.subcore_barrier()

    def g_start(buf, sem, chunk):
        pltpu.async_copy(y_hbm.at[idx_v.at[0, chunk]], buf, sem)

    def g_wait(buf, sem):
        pltpu.make_async_copy(y_hbm.at[pl.ds(0, CH)], buf, sem).wait()

    def s_start(buf, sem, chunk):
        pltpu.async_copy(buf, acc_sh.at[idx_v.at[1, chunk]], sem, add=True)

    def s_wait(buf, sem, chunk):
        pltpu.make_async_copy(buf, acc_sh.at[idx_v.at[1, chunk]], sem).wait()

    def seg_loop(g, carry):
        pltpu.sync_copy(src_hbm.at[wid, g], idx_v.at[0])
        pltpu.sync_copy(dst_hbm.at[wid, g], idx_v.at[1])
        g_start(rows_a, sem_ga, 0)
        g_start(rows_b, sem_gb, 1)

        def triple(j, inner):
            c0 = 3 * j
            g_wait(rows_a, sem_ga)
            s_start(rows_a, sem_sa, c0)

            @pl.when(j > 0)
            def _():
                s_wait(rows_c, sem_sc, c0 - 1)
            g_start(rows_c, sem_gc, c0 + 2)

            g_wait(rows_b, sem_gb)
            s_start(rows_b, sem_sb, c0 + 1)

            s_wait(rows_a, sem_sa, c0)
            g_start(rows_a, sem_ga, c0 + 3)

            g_wait(rows_c, sem_gc)
            s_start(rows_c, sem_sc, c0 + 2)

            s_wait(rows_b, sem_sb, c0 + 1)

            @pl.when(j < (SEGC - 1) // 3 - 1)
            def _():
                g_start(rows_b, sem_gb, c0 + 4)
            return inner
        lax.fori_loop(0, (SEGC - 1) // 3, triple, 0)

        g_wait(rows_a, sem_ga)
        s_start(rows_a, sem_sa, SEGC - 1)
        s_wait(rows_c, sem_sc, SEGC - 2)
        s_wait(rows_a, sem_sa, SEGC - 1)
        return carry
    lax.fori_loop(0, SEG, seg_loop, 0)

    plsc.subcore_barrier()
    pltpu.sync_copy(acc_sh.at[pl.ds(s * npt, npt)],
                    out_hbm.at[c, pl.ds(s * npt, npt)])


def _make_prop(n, d, n_pad):
    npt = n_pad // NS
    body = functools.partial(_prop_body, d, npt)
    return pl.kernel(
        body,
        jax.ShapeDtypeStruct((NC, n_pad, d), jnp.float32),
        mesh=plsc.VectorSubcoreMesh(core_axis_name="c", subcore_axis_name="s"),
        scratch_types=[
            pltpu.VMEM((2, SEGC, CH), jnp.int32),
            pltpu.VMEM((CH, d), jnp.float32),
            pltpu.VMEM((CH, d), jnp.float32),
            pltpu.VMEM((CH, d), jnp.float32),
            pltpu.VMEM((8, d), jnp.float32),
            pltpu.VMEM_SHARED((n_pad, d), jnp.float32),
            pltpu.SemaphoreType.DMA,
            pltpu.SemaphoreType.DMA,
            pltpu.SemaphoreType.DMA,
            pltpu.SemaphoreType.DMA,
            pltpu.SemaphoreType.DMA,
            pltpu.SemaphoreType.DMA,
        ],
    )


def _tc_a1_body(x_ref, w_ref, xw_ref):
    xw_ref[...] = jnp.dot(x_ref[...], w_ref[...],
                          preferred_element_type=jnp.float32)


def _tc_a2_body(deg0_ref, deg1_ref, xw_ref, y_ref, dinv_ref):
    dinv = lax.rsqrt(deg0_ref[...] + deg1_ref[...] + 1.0)
    y_ref[...] = xw_ref[...] * dinv[:, None]
    dinv_ref[...] = dinv


def _tc_b_body(p_ref, y1_ref, dinv_ref, b1_ref, y2_ref):
    dinv = dinv_ref[...][:, None]
    h = jnp.maximum(dinv * (p_ref[0] + p_ref[1] + y1_ref[...]) + b1_ref[...],
                    0.0)
    y2_ref[...] = h * dinv


def _tc_c_body(q_ref, y2_ref, dinv_ref, b2_ref, w2_ref, out_ref):
    g = dinv_ref[...][:, None] * (q_ref[0] + q_ref[1] + y2_ref[...])
    z = jnp.dot(g, w2_ref[...],
                preferred_element_type=jnp.float32) + b2_ref[...]
    m = jnp.max(z, axis=1, keepdims=True)
    lse = jnp.log(jnp.sum(jnp.exp(z - m), axis=1, keepdims=True)) + m
    out_ref[...] = z - lse


def kernel(x, edge_index, W1, b1, W2, b2):
    n, f = x.shape
    nh = W1.shape[1]
    nc = W2.shape[1]
    e = edge_index.shape[1]
    n_pad = ((n + NS * 16 - 1) // (NS * 16)) * (NS * 16)
    R = 1024
    grid = (n_pad // R,)

    ei = edge_index.astype(jnp.int32)
    src4 = ei[0].reshape(NC * NS, SEG, SEGC, CH)
    dst4 = ei[1].reshape(NC * NS, SEG, SEGC, CH)

    deg0, deg1 = _make_deg(n_pad)(dst4)

    xw = pl.pallas_call(
        _tc_a1_body,
        grid=grid,
        in_specs=[
            pl.BlockSpec((R, f), lambda i: (i, 0)),
            pl.BlockSpec((f, nh), lambda i: (0, 0)),
        ],
        out_specs=pl.BlockSpec((R, nh), lambda i: (i, 0)),
        out_shape=jax.ShapeDtypeStruct((n, nh), jnp.float32),
    )(x, W1)

    y1, dinv = pl.pallas_call(
        _tc_a2_body,
        grid=grid,
        in_specs=[
            pl.BlockSpec((R,), lambda i: (i,)),
            pl.BlockSpec((R,), lambda i: (i,)),
            pl.BlockSpec((R, nh), lambda i: (i, 0)),
        ],
        out_specs=[
            pl.BlockSpec((R, nh), lambda i: (i, 0)),
            pl.BlockSpec((R,), lambda i: (i,)),
        ],
        out_shape=[
            jax.ShapeDtypeStruct((n, nh), jnp.float32),
            jax.ShapeDtypeStruct((n_pad,), jnp.float32),
        ],
    )(deg0, deg1, xw)

    p = _make_prop(n, nh, n_pad)(y1, src4, dst4)

    y2 = pl.pallas_call(
        _tc_b_body,
        grid=grid,
        in_specs=[
            pl.BlockSpec((NC, R, nh), lambda i: (0, i, 0)),
            pl.BlockSpec((R, nh), lambda i: (i, 0)),
            pl.BlockSpec((R,), lambda i: (i,)),
            pl.BlockSpec((1, nh), lambda i: (0, 0)),
        ],
        out_specs=pl.BlockSpec((R, nh), lambda i: (i, 0)),
        out_shape=jax.ShapeDtypeStruct((n, nh), jnp.float32),
    )(p, y1, dinv, b1.reshape(1, nh))

    q = _make_prop(n, nh, n_pad)(y2, src4, dst4)

    out = pl.pallas_call(
        _tc_c_body,
        grid=grid,
        in_specs=[
            pl.BlockSpec((NC, R, nh), lambda i: (0, i, 0)),
            pl.BlockSpec((R, nh), lambda i: (i, 0)),
            pl.BlockSpec((R,), lambda i: (i,)),
            pl.BlockSpec((1, nc), lambda i: (0, 0)),
            pl.BlockSpec((nh, nc), lambda i: (0, 0)),
        ],
        out_specs=pl.BlockSpec((R, nc), lambda i: (i, 0)),
        out_shape=jax.ShapeDtypeStruct((n, nc), jnp.float32),
    )(q, y2, dinv, b2.reshape(1, nc), W2)

    return out


# speedup vs baseline: 32.4418x; 1.0105x over previous
import functools

import jax
import jax.numpy as jnp
from jax import lax
from jax.experimental import pallas as pl
from jax.experimental.pallas import tpu as pltpu
from jax.experimental.pallas import tpu_sc as plsc

NC = 2
NS = 16
CH = 80
SEG = 5
SEGC = 25


def _deg_body(npt, dst_hbm, out0_hbm, out1_hbm, didx_v, ones_v, zeros_v,
              acc_sh):
    c = lax.axis_index("c")
    s = lax.axis_index("s")
    wid = c * NS + s

    def fill_ones(i, carry):
        ones_v[pl.ds(i * 16, 16)] = jnp.full((16,), 1.0, jnp.float32)
        return carry
    lax.fori_loop(0, CH // 16, fill_ones, 0)

    def fill_zeros(i, carry):
        zeros_v[pl.ds(i * 16, 16)] = jnp.zeros((16,), jnp.float32)
        return carry
    lax.fori_loop(0, 640 // 16, fill_zeros, 0)

    pltpu.sync_copy(zeros_v.at[pl.ds(0, npt)], acc_sh.at[pl.ds(s * npt, npt)])
    plsc.subcore_barrier()

    def seg_loop(g, carry):
        pltpu.sync_copy(dst_hbm.at[wid, g], didx_v)

        def fire(j, inner):
            pltpu.sync_copy(ones_v, acc_sh.at[didx_v.at[j]], add=True)
            return inner
        lax.fori_loop(0, SEGC, fire, 0)
        return carry
    lax.fori_loop(0, SEG, seg_loop, 0)

    plsc.subcore_barrier()

    @pl.when(c == 0)
    def _():
        pltpu.sync_copy(acc_sh.at[pl.ds(s * npt, npt)],
                        out0_hbm.at[pl.ds(s * npt, npt)])

    @pl.when(c == 1)
    def _():
        pltpu.sync_copy(acc_sh.at[pl.ds(s * npt, npt)],
                        out1_hbm.at[pl.ds(s * npt, npt)])


def _make_deg(n_pad):
    npt = n_pad // NS
    body = functools.partial(_deg_body, npt)
    return pl.kernel(
        body,
        [jax.ShapeDtypeStruct((n_pad,), jnp.float32),
         jax.ShapeDtypeStruct((n_pad,), jnp.float32)],
        mesh=plsc.VectorSubcoreMesh(core_axis_name="c", subcore_axis_name="s"),
        scratch_types=[
            pltpu.VMEM((SEGC, CH), jnp.int32),
            pltpu.VMEM((CH,), jnp.float32),
            pltpu.VMEM((640,), jnp.float32),
            pltpu.VMEM_SHARED((n_pad,), jnp.float32),
        ],
    )


def _prop_body(d, npt, y_hbm, src_hbm, dst_hbm, out_hbm, idx_v,
               rows_a, rows_b, rows_c, acc_sh,
               sem_ga, sem_gb, sem_gc, sem_sa, sem_sb, sem_sc):
    c = lax.axis_index("c")
    s = lax.axis_index("s")
    wid = c * NS + s

    nz = d // 16

    def fill_zeros(i, carry):
        rows_a[i // nz, pl.ds((i % nz) * 16, 16)] = jnp.zeros((16,),
                                                              jnp.float32)
        return carry
    lax.fori_loop(0, CH * nz, fill_zeros, 0)

    def zero_acc(j, carry):
        pltpu.sync_copy(rows_a, acc_sh.at[pl.ds(s * npt + j * CH, CH)])
        return carry
    lax.fori_loop(0, npt // CH, zero_acc, 0)
    plsc.subcore_barrier()

    def g_start(buf, sem, chunk):
        pltpu.async_copy(y_hbm.at[idx_v.at[0, chunk]], buf, sem)

    def g_wait(buf, sem):
        pltpu.make_async_copy(y_hbm.at[pl.ds(0, CH)], buf, sem).wait()

    def s_start(buf, sem, chunk):
        pltpu.async_copy(buf, acc_sh.at[idx_v.at[1, chunk]], sem, add=True)

    def s_wait(buf, sem, chunk):
        pltpu.make_async_copy(buf, acc_sh.at[idx_v.at[1, chunk]], sem).wait()

    def seg_loop(g, carry):
        pltpu.sync_copy(src_hbm.at[wid, g], idx_v.at[0])
        pltpu.sync_copy(dst_hbm.at[wid, g], idx_v.at[1])
        g_start(rows_a, sem_ga, 0)
        g_start(rows_b, sem_gb, 1)

        def triple(j, inner):
            c0 = 3 * j
            g_wait(rows_a, sem_ga)
            s_start(rows_a, sem_sa, c0)

            @pl.when(j > 0)
            def _():
                s_wait(rows_c, sem_sc, c0 - 1)
            g_start(rows_c, sem_gc, c0 + 2)

            g_wait(rows_b, sem_gb)
            s_start(rows_b, sem_sb, c0 + 1)

            s_wait(rows_a, sem_sa, c0)
            g_start(rows_a, sem_ga, c0 + 3)

            g_wait(rows_c, sem_gc)
            s_start(rows_c, sem_sc, c0 + 2)

            s_wait(rows_b, sem_sb, c0 + 1)

            @pl.when(j < (SEGC - 1) // 3 - 1)
            def _():
                g_start(rows_b, sem_gb, c0 + 4)
            return inner
        lax.fori_loop(0, (SEGC - 1) // 3, triple, 0)

        g_wait(rows_a, sem_ga)
        s_start(rows_a, sem_sa, SEGC - 1)
        s_wait(rows_c, sem_sc, SEGC - 2)
        s_wait(rows_a, sem_sa, SEGC - 1)
        return carry
    lax.fori_loop(0, SEG, seg_loop, 0)

    plsc.subcore_barrier()
    pltpu.sync_copy(acc_sh.at[pl.ds(s * npt, npt)],
                    out_hbm.at[c, pl.ds(s * npt, npt)])


def _make_prop(n, d, n_pad):
    npt = n_pad // NS
    body = functools.partial(_prop_body, d, npt)
    return pl.kernel(
        body,
        jax.ShapeDtypeStruct((NC, n_pad, d), jnp.float32),
        mesh=plsc.VectorSubcoreMesh(core_axis_name="c", subcore_axis_name="s"),
        scratch_types=[
            pltpu.VMEM((2, SEGC, CH), jnp.int32),
            pltpu.VMEM((CH, d), jnp.float32),
            pltpu.VMEM((CH, d), jnp.float32),
            pltpu.VMEM((CH, d), jnp.float32),
            pltpu.VMEM_SHARED((n_pad, d), jnp.float32),
            pltpu.SemaphoreType.DMA,
            pltpu.SemaphoreType.DMA,
            pltpu.SemaphoreType.DMA,
            pltpu.SemaphoreType.DMA,
            pltpu.SemaphoreType.DMA,
            pltpu.SemaphoreType.DMA,
        ],
    )


def _tc_a1_body(x_ref, w_ref, xw_ref):
    xw_ref[...] = jnp.dot(x_ref[...], w_ref[...],
                          preferred_element_type=jnp.float32)


def _tc_a2_body(deg0_ref, deg1_ref, xw_ref, y_ref, dinv_ref):
    dinv = lax.rsqrt(deg0_ref[...] + deg1_ref[...] + 1.0)
    y_ref[...] = xw_ref[...] * dinv[:, None]
    dinv_ref[...] = dinv


def _tc_b_body(p_ref, y1_ref, dinv_ref, b1_ref, y2_ref):
    dinv = dinv_ref[...][:, None]
    h = jnp.maximum(dinv * (p_ref[0] + p_ref[1] + y1_ref[...]) + b1_ref[...],
                    0.0)
    y2_ref[...] = h * dinv


def _tc_c_body(q_ref, y2_ref, dinv_ref, b2_ref, w2_ref, out_ref):
    g = dinv_ref[...][:, None] * (q_ref[0] + q_ref[1] + y2_ref[...])
    z = jnp.dot(g, w2_ref[...],
                preferred_element_type=jnp.float32) + b2_ref[...]
    m = jnp.max(z, axis=1, keepdims=True)
    lse = jnp.log(jnp.sum(jnp.exp(z - m), axis=1, keepdims=True)) + m
    out_ref[...] = z - lse


def kernel(x, edge_index, W1, b1, W2, b2):
    n, f = x.shape
    nh = W1.shape[1]
    nc = W2.shape[1]
    e = edge_index.shape[1]
    n_pad = ((n + NS * 16 - 1) // (NS * 16)) * (NS * 16)
    R = 1024
    grid = (n_pad // R,)

    ei = edge_index.astype(jnp.int32)
    src4 = ei[0].reshape(NC * NS, SEG, SEGC, CH)
    dst4 = ei[1].reshape(NC * NS, SEG, SEGC, CH)

    deg0, deg1 = _make_deg(n_pad)(dst4)

    xw = pl.pallas_call(
        _tc_a1_body,
        grid=grid,
        in_specs=[
            pl.BlockSpec((R, f), lambda i: (i, 0)),
            pl.BlockSpec((f, nh), lambda i: (0, 0)),
        ],
        out_specs=pl.BlockSpec((R, nh), lambda i: (i, 0)),
        out_shape=jax.ShapeDtypeStruct((n, nh), jnp.float32),
    )(x, W1)

    y1, dinv = pl.pallas_call(
        _tc_a2_body,
        grid=grid,
        in_specs=[
            pl.BlockSpec((R,), lambda i: (i,)),
            pl.BlockSpec((R,), lambda i: (i,)),
            pl.BlockSpec((R, nh), lambda i: (i, 0)),
        ],
        out_specs=[
            pl.BlockSpec((R, nh), lambda i: (i, 0)),
            pl.BlockSpec((R,), lambda i: (i,)),
        ],
        out_shape=[
            jax.ShapeDtypeStruct((n, nh), jnp.float32),
            jax.ShapeDtypeStruct((n_pad,), jnp.float32),
        ],
    )(deg0, deg1, xw)

    p = _make_prop(n, nh, n_pad)(y1, src4, dst4)

    y2 = pl.pallas_call(
        _tc_b_body,
        grid=grid,
        in_specs=[
            pl.BlockSpec((NC, R, nh), lambda i: (0, i, 0)),
            pl.BlockSpec((R, nh), lambda i: (i, 0)),
            pl.BlockSpec((R,), lambda i: (i,)),
            pl.BlockSpec((1, nh), lambda i: (0, 0)),
        ],
        out_specs=pl.BlockSpec((R, nh), lambda i: (i, 0)),
        out_shape=jax.ShapeDtypeStruct((n, nh), jnp.float32),
    )(p, y1, dinv, b1.reshape(1, nh))

    q = _make_prop(n, nh, n_pad)(y2, src4, dst4)

    out = pl.pallas_call(
        _tc_c_body,
        grid=grid,
        in_specs=[
            pl.BlockSpec((NC, R, nh), lambda i: (0, i, 0)),
            pl.BlockSpec((R, nh), lambda i: (i, 0)),
            pl.BlockSpec((R,), lambda i: (i,)),
            pl.BlockSpec((1, nc), lambda i: (0, 0)),
            pl.BlockSpec((nh, nc), lambda i: (0, 0)),
        ],
        out_specs=pl.BlockSpec((R, nc), lambda i: (i, 0)),
        out_shape=jax.ShapeDtypeStruct((n, nc), jnp.float32),
    )(q, y2, dinv, b2.reshape(1, nc), W2)

    return out


# speedup vs baseline: 34.9051x; 1.0759x over previous
import functools

import jax
import jax.numpy as jnp
from jax import lax
from jax.experimental import pallas as pl
from jax.experimental.pallas import tpu as pltpu
from jax.experimental.pallas import tpu_sc as plsc

NC = 2
NS = 16
CH = 80
SEG = 5
SEGC = 25


def _deg_body(npt, dst_hbm, out0_hbm, out1_hbm, didx_v, ones_v, zeros_v,
              acc_sh):
    c = lax.axis_index("c")
    s = lax.axis_index("s")
    wid = c * NS + s

    def fill_ones(i, carry):
        ones_v[pl.ds(i * 16, 16)] = jnp.full((16,), 1.0, jnp.float32)
        return carry
    lax.fori_loop(0, CH // 16, fill_ones, 0)

    def fill_zeros(i, carry):
        zeros_v[pl.ds(i * 16, 16)] = jnp.zeros((16,), jnp.float32)
        return carry
    lax.fori_loop(0, 640 // 16, fill_zeros, 0)

    pltpu.sync_copy(zeros_v.at[pl.ds(0, npt)], acc_sh.at[pl.ds(s * npt, npt)])
    plsc.subcore_barrier()

    def seg_loop(g, carry):
        pltpu.sync_copy(dst_hbm.at[wid, g], didx_v)

        def fire(j, inner):
            pltpu.sync_copy(ones_v, acc_sh.at[didx_v.at[j]], add=True)
            return inner
        lax.fori_loop(0, SEGC, fire, 0)
        return carry
    lax.fori_loop(0, SEG, seg_loop, 0)

    plsc.subcore_barrier()

    @pl.when(c == 0)
    def _():
        pltpu.sync_copy(acc_sh.at[pl.ds(s * npt, npt)],
                        out0_hbm.at[pl.ds(s * npt, npt)])

    @pl.when(c == 1)
    def _():
        pltpu.sync_copy(acc_sh.at[pl.ds(s * npt, npt)],
                        out1_hbm.at[pl.ds(s * npt, npt)])


def _make_deg(n_pad):
    npt = n_pad // NS
    body = functools.partial(_deg_body, npt)
    return pl.kernel(
        body,
        [jax.ShapeDtypeStruct((n_pad,), jnp.float32),
         jax.ShapeDtypeStruct((n_pad,), jnp.float32)],
        mesh=plsc.VectorSubcoreMesh(core_axis_name="c", subcore_axis_name="s"),
        scratch_types=[
            pltpu.VMEM((SEGC, CH), jnp.int32),
            pltpu.VMEM((CH,), jnp.float32),
            pltpu.VMEM((640,), jnp.float32),
            pltpu.VMEM_SHARED((n_pad,), jnp.float32),
        ],
    )


def _prop_body(d, npt, y_hbm, src_hbm, dst_hbm, out_hbm, idx_v,
               rows_a, rows_b, rows_c, acc_sh,
               sem_ga, sem_gb, sem_gc, sem_sa, sem_sb, sem_sc):
    c = lax.axis_index("c")
    s = lax.axis_index("s")
    wid = c * NS + s

    nz = d // 16

    def fill_zeros(i, carry):
        rows_a[i // nz, pl.ds((i % nz) * 16, 16)] = jnp.zeros((16,),
                                                              jnp.float32)
        return carry
    lax.fori_loop(0, CH * nz, fill_zeros, 0)

    def zero_acc(j, carry):
        pltpu.sync_copy(rows_a, acc_sh.at[pl.ds(s * npt + j * CH, CH)])
        return carry
    lax.fori_loop(0, npt // CH, zero_acc, 0)
    plsc.subcore_barrier()

    def g_start(buf, sem, chunk):
        pltpu.async_copy(y_hbm.at[idx_v.at[0, chunk]], buf, sem)

    def g_wait(buf, sem):
        pltpu.make_async_copy(y_hbm.at[pl.ds(0, CH)], buf, sem).wait()

    def s_start(buf, sem, chunk):
        pltpu.async_copy(buf, acc_sh.at[idx_v.at[1, chunk]], sem, add=True)

    def s_wait(buf, sem, chunk):
        pltpu.make_async_copy(buf, acc_sh.at[idx_v.at[1, chunk]], sem).wait()

    def seg_loop(g, carry):
        pltpu.sync_copy(src_hbm.at[wid, g], idx_v.at[0])
        pltpu.sync_copy(dst_hbm.at[wid, g], idx_v.at[1])
        g_start(rows_a, sem_ga, 0)
        g_start(rows_b, sem_gb, 1)

        def triple(j, inner):
            c0 = 3 * j
            g_wait(rows_a, sem_ga)
            s_start(rows_a, sem_sa, c0)

            @pl.when(j > 0)
            def _():
                s_wait(rows_c, sem_sc, c0 - 1)
            g_start(rows_c, sem_gc, c0 + 2)

            g_wait(rows_b, sem_gb)
            s_start(rows_b, sem_sb, c0 + 1)

            s_wait(rows_a, sem_sa, c0)
            g_start(rows_a, sem_ga, c0 + 3)

            g_wait(rows_c, sem_gc)
            s_start(rows_c, sem_sc, c0 + 2)

            s_wait(rows_b, sem_sb, c0 + 1)

            @pl.when(j < (SEGC - 1) // 3 - 1)
            def _():
                g_start(rows_b, sem_gb, c0 + 4)
            return inner
        lax.fori_loop(0, (SEGC - 1) // 3, triple, 0)

        g_wait(rows_a, sem_ga)
        s_start(rows_a, sem_sa, SEGC - 1)
        s_wait(rows_c, sem_sc, SEGC - 2)
        s_wait(rows_a, sem_sa, SEGC - 1)
        return carry
    lax.fori_loop(0, SEG, seg_loop, 0)

    plsc.subcore_barrier()
    pltpu.sync_copy(acc_sh.at[pl.ds(s * npt, npt)],
                    out_hbm.at[c, pl.ds(s * npt, npt)])


def _make_prop(n, d, n_pad, tc_tiling=True):
    npt = n_pad // NS
    body = functools.partial(_prop_body, d, npt)
    return pl.kernel(
        body,
        jax.ShapeDtypeStruct((NC, n_pad, d), jnp.float32),
        mesh=plsc.VectorSubcoreMesh(core_axis_name="c", subcore_axis_name="s"),
        compiler_params=pltpu.CompilerParams(use_tc_tiling_on_sc=tc_tiling),
        scratch_types=[
            pltpu.VMEM((2, SEGC, CH), jnp.int32),
            pltpu.VMEM((CH, d), jnp.float32),
            pltpu.VMEM((CH, d), jnp.float32),
            pltpu.VMEM((CH, d), jnp.float32),
            pltpu.VMEM_SHARED((n_pad, d), jnp.float32),
            pltpu.SemaphoreType.DMA,
            pltpu.SemaphoreType.DMA,
            pltpu.SemaphoreType.DMA,
            pltpu.SemaphoreType.DMA,
            pltpu.SemaphoreType.DMA,
            pltpu.SemaphoreType.DMA,
        ],
    )


def _tc_a1_body(x_ref, w_ref, xw_ref):
    xw_ref[...] = jnp.dot(x_ref[...], w_ref[...],
                          preferred_element_type=jnp.float32)


def _tc_a2_body(deg0_ref, deg1_ref, xw_ref, y_ref, dinv_ref):
    dinv = lax.rsqrt(deg0_ref[...] + deg1_ref[...] + 1.0)
    y_ref[...] = xw_ref[...] * dinv[:, None]
    dinv_ref[...] = dinv


def _tc_b_body(p_ref, y1_ref, dinv_ref, b1_ref, w2_ref, y2_ref):
    dinv = dinv_ref[...][:, None]
    h = jnp.maximum(dinv * (p_ref[0] + p_ref[1] + y1_ref[...]) + b1_ref[...],
                    0.0)
    y2_ref[...] = jnp.dot(h, w2_ref[...],
                          preferred_element_type=jnp.float32) * dinv


def _tc_c_body(q_ref, y2_ref, dinv_ref, b2_ref, out_ref):
    z = (dinv_ref[...][:, None] * (q_ref[0] + q_ref[1] + y2_ref[...])
         + b2_ref[...])
    m = jnp.max(z, axis=1, keepdims=True)
    lse = jnp.log(jnp.sum(jnp.exp(z - m), axis=1, keepdims=True)) + m
    out_ref[...] = z - lse


def kernel(x, edge_index, W1, b1, W2, b2):
    n, f = x.shape
    nh = W1.shape[1]
    nc = W2.shape[1]
    e = edge_index.shape[1]
    n_pad = ((n + NS * 16 - 1) // (NS * 16)) * (NS * 16)
    R = 1024
    grid = (n_pad // R,)

    ei = edge_index.astype(jnp.int32)
    src4 = ei[0].reshape(NC * NS, SEG, SEGC, CH)
    dst4 = ei[1].reshape(NC * NS, SEG, SEGC, CH)

    deg0, deg1 = _make_deg(n_pad)(dst4)

    xw = pl.pallas_call(
        _tc_a1_body,
        grid=grid,
        in_specs=[
            pl.BlockSpec((R, f), lambda i: (i, 0)),
            pl.BlockSpec((f, nh), lambda i: (0, 0)),
        ],
        out_specs=pl.BlockSpec((R, nh), lambda i: (i, 0)),
        out_shape=jax.ShapeDtypeStruct((n, nh), jnp.float32),
    )(x, W1)

    y1, dinv = pl.pallas_call(
        _tc_a2_body,
        grid=grid,
        in_specs=[
            pl.BlockSpec((R,), lambda i: (i,)),
            pl.BlockSpec((R,), lambda i: (i,)),
            pl.BlockSpec((R, nh), lambda i: (i, 0)),
        ],
        out_specs=[
            pl.BlockSpec((R, nh), lambda i: (i, 0)),
            pl.BlockSpec((R,), lambda i: (i,)),
        ],
        out_shape=[
            jax.ShapeDtypeStruct((n, nh), jnp.float32),
            jax.ShapeDtypeStruct((n_pad,), jnp.float32),
        ],
    )(deg0, deg1, xw)

    p = _make_prop(n, nh, n_pad)(y1, src4, dst4)

    y2 = pl.pallas_call(
        _tc_b_body,
        grid=grid,
        in_specs=[
            pl.BlockSpec((NC, R, nh), lambda i: (0, i, 0)),
            pl.BlockSpec((R, nh), lambda i: (i, 0)),
            pl.BlockSpec((R,), lambda i: (i,)),
            pl.BlockSpec((1, nh), lambda i: (0, 0)),
            pl.BlockSpec((nh, nc), lambda i: (0, 0)),
        ],
        out_specs=pl.BlockSpec((R, nc), lambda i: (i, 0)),
        out_shape=jax.ShapeDtypeStruct((n, nc), jnp.float32),
    )(p, y1, dinv, b1.reshape(1, nh), W2)

    q = _make_prop(n, nc, n_pad, tc_tiling=False)(y2, src4, dst4)

    out = pl.pallas_call(
        _tc_c_body,
        grid=grid,
        in_specs=[
            pl.BlockSpec((NC, R, nc), lambda i: (0, i, 0)),
            pl.BlockSpec((R, nc), lambda i: (i, 0)),
            pl.BlockSpec((R,), lambda i: (i,)),
            pl.BlockSpec((1, nc), lambda i: (0, 0)),
        ],
        out_specs=pl.BlockSpec((R, nc), lambda i: (i, 0)),
        out_shape=jax.ShapeDtypeStruct((n, nc), jnp.float32),
    )(q, y2, dinv, b2.reshape(1, nc))

    return out
